# Initial kernel scaffold; baseline (speedup 1.0000x reference)
import jax
import jax.numpy as jnp
from jax.experimental import pallas as pl


def kernel(x, edge_index, gamma, beta, W1, b1, tag1_W, tag1_b, W2, b2, tag2_W, tag2_b, W5, b5, W6, b6, Wc, bc):
    raise NotImplementedError("write your pallas kernel here")


# speedup vs baseline: 1.4475x; 1.4475x over previous
import functools

import jax
import jax.numpy as jnp
from jax.experimental import pallas as pl

N = 10000
D = 128
BN = 1000

_INV_SQRT2 = 0.7071067811865476


def _gelu(t):
    return 0.5 * t * (1.0 + jax.lax.erf(t * _INV_SQRT2))


def _linear_kernel(x_ref, w_ref, b_ref, o_ref, *, act):
    y = jnp.dot(x_ref[...], w_ref[...], preferred_element_type=jnp.float32)
    y = y + b_ref[...]
    if act == "gelu":
        y = _gelu(y)
    o_ref[...] = y


def _linear(x, W, b, act="none"):
    n, d = x.shape
    do = W.shape[1]
    return pl.pallas_call(
        functools.partial(_linear_kernel, act=act),
        grid=(n // BN,),
        in_specs=[
            pl.BlockSpec((BN, d), lambda i: (i, 0)),
            pl.BlockSpec((d, do), lambda i: (0, 0)),
            pl.BlockSpec((1, do), lambda i: (0, 0)),
        ],
        out_specs=pl.BlockSpec((BN, do), lambda i: (i, 0)),
        out_shape=jax.ShapeDtypeStruct((n, do), jnp.float32),
    )(x, W, b.reshape(1, do))


def _stats_kernel(x_ref, s_ref, q_ref):
    i = pl.program_id(0)

    @pl.when(i == 0)
    def _init():
        s_ref[...] = jnp.zeros_like(s_ref)
        q_ref[...] = jnp.zeros_like(q_ref)

    xb = x_ref[...]
    s_ref[...] += jnp.sum(xb, axis=0, keepdims=True)
    q_ref[...] += jnp.sum(xb * xb, axis=0, keepdims=True)


def _bn_stats(x):
    n, d = x.shape
    s, q = pl.pallas_call(
        _stats_kernel,
        grid=(n // BN,),
        in_specs=[pl.BlockSpec((BN, d), lambda i: (i, 0))],
        out_specs=[
            pl.BlockSpec((1, d), lambda i: (0, 0)),
            pl.BlockSpec((1, d), lambda i: (0, 0)),
        ],
        out_shape=[
            jax.ShapeDtypeStruct((1, d), jnp.float32),
            jax.ShapeDtypeStruct((1, d), jnp.float32),
        ],
    )(x)
    mean = s / n
    var = q / n - mean * mean
    return mean, var


def _bn_kernel(x_ref, sc_ref, sh_ref, o_ref):
    o_ref[...] = x_ref[...] * sc_ref[...] + sh_ref[...]


def _bn_apply(x, scale, shift):
    n, d = x.shape
    return pl.pallas_call(
        _bn_kernel,
        grid=(n // BN,),
        in_specs=[
            pl.BlockSpec((BN, d), lambda i: (i, 0)),
            pl.BlockSpec((1, d), lambda i: (0, 0)),
            pl.BlockSpec((1, d), lambda i: (0, 0)),
        ],
        out_specs=pl.BlockSpec((BN, d), lambda i: (i, 0)),
        out_shape=jax.ShapeDtypeStruct((n, d), jnp.float32),
    )(x, scale, shift)


def _head_kernel(a_ref, w6_ref, b6_ref, wc_ref, bc_ref, sm_ref, cl_ref):
    a = jnp.dot(a_ref[...], w6_ref[...], preferred_element_type=jnp.float32)
    a = a + b6_ref[...]
    g = _gelu(a)
    cl_ref[...] = jnp.dot(g, wc_ref[...], preferred_element_type=jnp.float32) + bc_ref[...]
    m = jnp.max(a, axis=-1, keepdims=True)
    e = jnp.exp(a - m)
    sm_ref[...] = e / jnp.sum(e, axis=-1, keepdims=True)


def _head(a, W6, b6, Wc, bc):
    n, d = a.shape
    do = W6.shape[1]
    dc = Wc.shape[1]
    return pl.pallas_call(
        _head_kernel,
        grid=(n // BN,),
        in_specs=[
            pl.BlockSpec((BN, d), lambda i: (i, 0)),
            pl.BlockSpec((d, do), lambda i: (0, 0)),
            pl.BlockSpec((1, do), lambda i: (0, 0)),
            pl.BlockSpec((do, dc), lambda i: (0, 0)),
            pl.BlockSpec((1, dc), lambda i: (0, 0)),
        ],
        out_specs=[
            pl.BlockSpec((BN, do), lambda i: (i, 0)),
            pl.BlockSpec((BN, dc), lambda i: (i, 0)),
        ],
        out_shape=[
            jax.ShapeDtypeStruct((n, do), jnp.float32),
            jax.ShapeDtypeStruct((n, dc), jnp.float32),
        ],
    )(a, W6, b6.reshape(1, do), Wc, bc.reshape(1, dc))


def _propagate(u, row, col):
    return jax.ops.segment_sum(u[row], col, num_segments=u.shape[0])


def kernel(x, edge_index, gamma, beta, W1, b1, tag1_W, tag1_b, W2, b2,
           tag2_W, tag2_b, W5, b5, W6, b6, Wc, bc):
    row, col = edge_index[0], edge_index[1]

    deg = jax.ops.segment_sum(jnp.ones((row.shape[0],), jnp.float32), col,
                              num_segments=N)
    dis = jnp.where(deg > 0, jax.lax.rsqrt(deg), 0.0)
    dis = dis[:, None]

    mean, var = _bn_stats(x)
    scale = gamma.reshape(1, D) / jnp.sqrt(var + 1e-5)
    shift = beta.reshape(1, D) - mean * scale
    xn = _bn_apply(x, scale, shift)

    h = _linear(xn, W1, b1, act="gelu")

    def tagconv(h, Ws, b):
        out = _linear(h, Ws[0], b)
        u = h * dis
        for k in range(1, Ws.shape[0]):
            v = _propagate(u, row, col)
            out = out + _linear(v * dis, Ws[k], jnp.zeros_like(b))
            u = v * (dis * dis)
        return out

    h = jax.nn.gelu(tagconv(h, tag1_W, tag1_b), approximate=False)
    h = _linear(h, W2, b2, act="gelu")
    h = jax.nn.gelu(tagconv(h, tag2_W, tag2_b), approximate=False)

    a = jnp.concatenate([xn, h], axis=1)
    a = _linear(a, W5, b5, act="gelu")
    sm, cl = _head(a, W6, b6, Wc, bc)
    return (sm, cl)


# speedup vs baseline: 6.1045x; 4.2174x over previous
import functools

import jax
import jax.numpy as jnp
from jax import lax
from jax.experimental import pallas as pl
from jax.experimental.pallas import tpu as pltpu
from jax.experimental.pallas import tpu_sc as plsc

N = 10000
NP = 10240
D = 128
E = 320000
NC, NS = 2, 16
NW = NC * NS
C = 128
NCH = 80
EP = NW * NCH * C
NZB = 10048
RPT = NP // NS

BN = 1280
_INV_SQRT2 = 0.7071067811865476


def _gelu(t):
    return 0.5 * t * (1.0 + jax.lax.erf(t * _INV_SQRT2))


DH = D // 2
NCH2 = EP // (NS * C)

_MESH = plsc.VectorSubcoreMesh(core_axis_name="c", subcore_axis_name="s")


@functools.partial(
    pl.kernel,
    out_type=jax.ShapeDtypeStruct((NC, NP, DH), jnp.float32),
    mesh=_MESH,
    compiler_params=pltpu.CompilerParams(use_tc_tiling_on_sc=False),
    scratch_types=[
        pltpu.VMEM((NCH2, C), jnp.int32),
        pltpu.VMEM((NCH2, C), jnp.int32),
        pltpu.VMEM((C, DH), jnp.float32),
        pltpu.VMEM((C, DH), jnp.float32),
        pltpu.VMEM_SHARED((NP, DH), jnp.float32),
        pltpu.SemaphoreType.DMA,
        pltpu.SemaphoreType.DMA,
    ],
)
def _sc_prop_feat(u_hbm, row_hbm, col_hbm, out_hbm, rowi, coli, buf0, buf1,
                  ysh, g0, g1):
    cid = lax.axis_index("c")
    sid = lax.axis_index("s")
    base = sid * RPT
    for t in range(RPT // C):
        pltpu.sync_copy(u_hbm.at[pl.ds(NZB, C)],
                        ysh.at[pl.ds(base + t * C, C)])
    pltpu.sync_copy(row_hbm.at[cid, sid], rowi)
    pltpu.sync_copy(col_hbm.at[sid], coli)
    plsc.subcore_barrier()

    def _wait(buf, sem):
        pltpu.make_async_copy(u_hbm.at[rowi.at[0]], buf, sem).wait()

    pltpu.async_copy(u_hbm.at[rowi.at[0]], buf0, g0)

    def pair(g, carry):
        j = 2 * g
        pltpu.async_copy(u_hbm.at[rowi.at[j + 1]], buf1, g1)
        _wait(buf0, g0)
        pltpu.sync_copy(buf0, ysh.at[coli.at[j]], add=True)
        pltpu.async_copy(u_hbm.at[rowi.at[j + 2]], buf0, g0)
        _wait(buf1, g1)
        pltpu.sync_copy(buf1, ysh.at[coli.at[j + 1]], add=True)
        return carry

    lax.fori_loop(0, NCH2 // 2 - 1, pair, 0)
    pltpu.async_copy(u_hbm.at[rowi.at[NCH2 - 1]], buf1, g1)
    _wait(buf0, g0)
    pltpu.sync_copy(buf0, ysh.at[coli.at[NCH2 - 2]], add=True)
    _wait(buf1, g1)
    pltpu.sync_copy(buf1, ysh.at[coli.at[NCH2 - 1]], add=True)

    plsc.subcore_barrier()
    pltpu.sync_copy(ysh.at[pl.ds(base, RPT)],
                    out_hbm.at[cid, pl.ds(base, RPT)])


@functools.partial(
    pl.kernel,
    out_type=jax.ShapeDtypeStruct((NC, NP, 16), jnp.float32),
    mesh=_MESH,
    compiler_params=pltpu.CompilerParams(use_tc_tiling_on_sc=False),
    scratch_types=[
        pltpu.VMEM((NCH, C), jnp.int32),
        pltpu.VMEM((NCH, C), jnp.int32),
        pltpu.VMEM((C, 16), jnp.float32),
        pltpu.VMEM((C, 16), jnp.float32),
        pltpu.VMEM_SHARED((NP, 16), jnp.float32),
        pltpu.SemaphoreType.DMA,
        pltpu.SemaphoreType.DMA,
    ],
)
def _sc_prop_deg(u_hbm, row_hbm, col_hbm, out_hbm, rowi, coli, buf0, buf1,
                 ysh, g0, g1):
    cid = lax.axis_index("c")
    sid = lax.axis_index("s")
    wid = cid * NS + sid
    base = sid * RPT
    for t in range(RPT // C):
        pltpu.sync_copy(u_hbm.at[pl.ds(NZB, C)],
                        ysh.at[pl.ds(base + t * C, C)])
    pltpu.sync_copy(row_hbm.at[wid], rowi)
    pltpu.sync_copy(col_hbm.at[wid], coli)
    plsc.subcore_barrier()

    def _wait(buf, sem):
        pltpu.make_async_copy(u_hbm.at[rowi.at[0]], buf, sem).wait()

    pltpu.async_copy(u_hbm.at[rowi.at[0]], buf0, g0)

    def pair(g, carry):
        j = 2 * g
        pltpu.async_copy(u_hbm.at[rowi.at[j + 1]], buf1, g1)
        _wait(buf0, g0)
        pltpu.sync_copy(buf0, ysh.at[coli.at[j]], add=True)
        pltpu.async_copy(u_hbm.at[rowi.at[j + 2]], buf0, g0)
        _wait(buf1, g1)
        pltpu.sync_copy(buf1, ysh.at[coli.at[j + 1]], add=True)
        return carry

    lax.fori_loop(0, NCH // 2 - 1, pair, 0)
    pltpu.async_copy(u_hbm.at[rowi.at[NCH - 1]], buf1, g1)
    _wait(buf0, g0)
    pltpu.sync_copy(buf0, ysh.at[coli.at[NCH - 2]], add=True)
    _wait(buf1, g1)
    pltpu.sync_copy(buf1, ysh.at[coli.at[NCH - 1]], add=True)

    plsc.subcore_barrier()
    pltpu.sync_copy(ysh.at[pl.ds(base, RPT)],
                    out_hbm.at[cid, pl.ds(base, RPT)])


def _linear_kernel(x_ref, w_ref, b_ref, o_ref, *, act):
    y = jnp.dot(x_ref[...], w_ref[...], preferred_element_type=jnp.float32)
    y = y + b_ref[...]
    if act == "gelu":
        y = _gelu(y)
    o_ref[...] = y


def _linear(x, W, b, act="none"):
    n, d = x.shape
    do = W.shape[1]
    return pl.pallas_call(
        functools.partial(_linear_kernel, act=act),
        grid=(n // BN,),
        in_specs=[
            pl.BlockSpec((BN, d), lambda i: (i, 0)),
            pl.BlockSpec((d, do), lambda i: (0, 0)),
            pl.BlockSpec((1, do), lambda i: (0, 0)),
        ],
        out_specs=pl.BlockSpec((BN, do), lambda i: (i, 0)),
        out_shape=jax.ShapeDtypeStruct((n, do), jnp.float32),
    )(x, W, b.reshape(1, do))


def _stats_kernel(x_ref, s_ref, q_ref):
    i = pl.program_id(0)

    @pl.when(i == 0)
    def _init():
        s_ref[...] = jnp.zeros_like(s_ref)
        q_ref[...] = jnp.zeros_like(q_ref)

    xb = x_ref[...]
    s_ref[...] += jnp.sum(xb, axis=0, keepdims=True)
    q_ref[...] += jnp.sum(xb * xb, axis=0, keepdims=True)


def _bn_stats(x):
    n, d = x.shape
    s, q = pl.pallas_call(
        _stats_kernel,
        grid=(n // BN,),
        in_specs=[pl.BlockSpec((BN, d), lambda i: (i, 0))],
        out_specs=[
            pl.BlockSpec((1, d), lambda i: (0, 0)),
            pl.BlockSpec((1, d), lambda i: (0, 0)),
        ],
        out_shape=[
            jax.ShapeDtypeStruct((1, d), jnp.float32),
            jax.ShapeDtypeStruct((1, d), jnp.float32),
        ],
    )(x)
    mean = s / N
    var = q / N - mean * mean
    return mean, var


def _bn_kernel(x_ref, sc_ref, sh_ref, o_ref):
    o_ref[...] = x_ref[...] * sc_ref[...] + sh_ref[...]


def _bn_apply(x, scale, shift):
    n, d = x.shape
    return pl.pallas_call(
        _bn_kernel,
        grid=(n // BN,),
        in_specs=[
            pl.BlockSpec((BN, d), lambda i: (i, 0)),
            pl.BlockSpec((1, d), lambda i: (0, 0)),
            pl.BlockSpec((1, d), lambda i: (0, 0)),
        ],
        out_specs=pl.BlockSpec((BN, d), lambda i: (i, 0)),
        out_shape=jax.ShapeDtypeStruct((n, d), jnp.float32),
    )(x, scale, shift)


def _dis_kernel(da_ref, db_ref, o_ref):
    deg = da_ref[...] + db_ref[...]
    o_ref[...] = jnp.where(deg > 0, jax.lax.rsqrt(deg), 0.0)


def _dis(dega, degb):
    return pl.pallas_call(
        _dis_kernel,
        grid=(NP // BN,),
        in_specs=[
            pl.BlockSpec((BN, 16), lambda i: (i, 0)),
            pl.BlockSpec((BN, 16), lambda i: (i, 0)),
        ],
        out_specs=pl.BlockSpec((BN, 16), lambda i: (i, 0)),
        out_shape=jax.ShapeDtypeStruct((NP, 16), jnp.float32),
    )(dega, degb)


def _scale_u_kernel(h_ref, d_ref, u_ref):
    t = h_ref[...] * d_ref[...][:, :1]
    u_ref[0, :, :] = t[:, :DH]
    u_ref[1, :, :] = t[:, DH:]


def _scale_u(h, dis16):
    return pl.pallas_call(
        _scale_u_kernel,
        grid=(NP // BN,),
        in_specs=[
            pl.BlockSpec((BN, D), lambda i: (i, 0)),
            pl.BlockSpec((BN, 16), lambda i: (i, 0)),
        ],
        out_specs=pl.BlockSpec((2, BN, DH), lambda i: (0, i, 0)),
        out_shape=jax.ShapeDtypeStruct((2, NP, DH), jnp.float32),
    )(h, dis16)


def _make_u_kernel(y_ref, d_ref, u_ref):
    dd = d_ref[...][:, :1]
    dd2 = dd * dd
    u_ref[0, :, :] = y_ref[0, :, :] * dd2
    u_ref[1, :, :] = y_ref[1, :, :] * dd2


def _make_u(y, dis16):
    return pl.pallas_call(
        _make_u_kernel,
        grid=(NP // BN,),
        in_specs=[
            pl.BlockSpec((2, BN, DH), lambda i: (0, i, 0)),
            pl.BlockSpec((BN, 16), lambda i: (i, 0)),
        ],
        out_specs=pl.BlockSpec((2, BN, DH), lambda i: (0, i, 0)),
        out_shape=jax.ShapeDtypeStruct((2, NP, DH), jnp.float32),
    )(y, dis16)


def _acc_kernel(y_ref, d_ref, w_ref, a_ref, o_ref, *, last):
    dd = d_ref[...][:, :1]
    t = jnp.concatenate([y_ref[0, :, :], y_ref[1, :, :]], axis=-1) * dd
    y = a_ref[...] + jnp.dot(t, w_ref[...], preferred_element_type=jnp.float32)
    if last:
        y = _gelu(y)
    o_ref[...] = y


def _acc_step(y, dis16, W, acc, last=False):
    return pl.pallas_call(
        functools.partial(_acc_kernel, last=last),
        grid=(NP // BN,),
        in_specs=[
            pl.BlockSpec((2, BN, DH), lambda i: (0, i, 0)),
            pl.BlockSpec((BN, 16), lambda i: (i, 0)),
            pl.BlockSpec((D, D), lambda i: (0, 0)),
            pl.BlockSpec((BN, D), lambda i: (i, 0)),
        ],
        out_specs=pl.BlockSpec((BN, D), lambda i: (i, 0)),
        out_shape=jax.ShapeDtypeStruct((NP, D), jnp.float32),
    )(y, dis16, W, acc)


def _linear2_kernel(x_ref, h_ref, wa_ref, wb_ref, b_ref, o_ref):
    y = jnp.dot(x_ref[...], wa_ref[...], preferred_element_type=jnp.float32)
    y += jnp.dot(h_ref[...], wb_ref[...], preferred_element_type=jnp.float32)
    o_ref[...] = _gelu(y + b_ref[...])


def _linear2(x, h, Wa, Wb, b):
    do = Wa.shape[1]
    return pl.pallas_call(
        _linear2_kernel,
        grid=(NP // BN,),
        in_specs=[
            pl.BlockSpec((BN, D), lambda i: (i, 0)),
            pl.BlockSpec((BN, D), lambda i: (i, 0)),
            pl.BlockSpec((D, do), lambda i: (0, 0)),
            pl.BlockSpec((D, do), lambda i: (0, 0)),
            pl.BlockSpec((1, do), lambda i: (0, 0)),
        ],
        out_specs=pl.BlockSpec((BN, do), lambda i: (i, 0)),
        out_shape=jax.ShapeDtypeStruct((NP, do), jnp.float32),
    )(x, h, Wa, Wb, b.reshape(1, do))


def _head_kernel(a_ref, w6_ref, b6_ref, wc_ref, bc_ref, sm_ref, cl_ref):
    a = jnp.dot(a_ref[...], w6_ref[...], preferred_element_type=jnp.float32)
    a = a + b6_ref[...]
    g = _gelu(a)
    cl_ref[...] = jnp.dot(g, wc_ref[...],
                          preferred_element_type=jnp.float32) + bc_ref[...]
    m = jnp.max(a, axis=-1, keepdims=True)
    e = jnp.exp(a - m)
    sm_ref[...] = e / jnp.sum(e, axis=-1, keepdims=True)


def _head(a, W6, b6, Wc, bc):
    do = W6.shape[1]
    dc = Wc.shape[1]
    return pl.pallas_call(
        _head_kernel,
        grid=(NP // BN,),
        in_specs=[
            pl.BlockSpec((BN, D), lambda i: (i, 0)),
            pl.BlockSpec((D, do), lambda i: (0, 0)),
            pl.BlockSpec((1, do), lambda i: (0, 0)),
            pl.BlockSpec((do, dc), lambda i: (0, 0)),
            pl.BlockSpec((1, dc), lambda i: (0, 0)),
        ],
        out_specs=[
            pl.BlockSpec((BN, do), lambda i: (i, 0)),
            pl.BlockSpec((BN, dc), lambda i: (i, 0)),
        ],
        out_shape=[
            jax.ShapeDtypeStruct((NP, do), jnp.float32),
            jax.ShapeDtypeStruct((NP, dc), jnp.float32),
        ],
    )(a, W6, b6.reshape(1, do), Wc, bc.reshape(1, dc))


def _tagconv(h, Ws, b, dis16, rowf, colf):
    acc = _linear(h, Ws[0], b)
    u = _scale_u(h, dis16)
    for k in range(1, 4):
        y = _sc_prop_feat(u.reshape(2 * NP, DH), rowf, colf)
        if k < 3:
            u = _make_u(y, dis16)
            acc = _acc_step(y, dis16, Ws[k], acc)
        else:
            acc = _acc_step(y, dis16, Ws[k], acc, last=True)
    return acc


def kernel(x, edge_index, gamma, beta, W1, b1, tag1_W, tag1_b, W2, b2,
           tag2_W, tag2_b, W5, b5, W6, b6, Wc, bc):
    row, col = edge_index[0], edge_index[1]
    pad = EP - E
    rowv = jnp.pad(row, (0, pad), constant_values=NP - 1)
    colv = jnp.pad(col, (0, pad), constant_values=NP - 1)
    rowp = rowv.reshape(NW, NCH, C)
    colp = colv.reshape(NW, NCH, C)
    rowf = jnp.stack([rowv, rowv + NP]).reshape(NC, NS, NCH2, C)
    colf = colv.reshape(NS, NCH2, C)
    x_pad = jnp.pad(x, ((0, NP - N), (0, 0)))
    ones_pad = jnp.pad(jnp.ones((N, 16), jnp.float32), ((0, NP - N), (0, 0)))

    degp = _sc_prop_deg(ones_pad, rowp, colp)
    dis16 = _dis(degp[0], degp[1])

    mean, var = _bn_stats(x_pad)
    scale = gamma.reshape(1, D) / jnp.sqrt(var + 1e-5)
    shift = beta.reshape(1, D) - mean * scale
    xn = _bn_apply(x_pad, scale, shift)

    h = _linear(xn, W1, b1, act="gelu")
    h = _tagconv(h, tag1_W, tag1_b, dis16, rowf, colf)
    h = _linear(h, W2, b2, act="gelu")
    h = _tagconv(h, tag2_W, tag2_b, dis16, rowf, colf)

    a = _linear2(xn, h, W5[:D], W5[D:], b5)
    sm, cl = _head(a, W6, b6, Wc, bc)
    return (sm[:N], cl[:N])
